# baseline (speedup 1.0000x reference)
import functools

import jax
import jax.numpy as jnp
from jax import lax
from jax.experimental import pallas as pl
from jax.experimental.pallas import tpu as pltpu
from jax.experimental.pallas import tpu_sc as plsc

_INFO = plsc.get_sparse_core_info()
_NC = _INFO.num_cores
_NS = _INFO.num_subcores
_NW = _NC * _NS
_L = 128
_IOTA = tuple(range(16))


def _iota16():
    return jax.lax.iota(jnp.int32, 16)


def _pack_table_call(dim, vocab):
    n_cols = (vocab // _L)
    n_k = (n_cols + _NW - 1) // _NW
    n_k2 = n_k + (n_k % 2)
    mesh = plsc.VectorSubcoreMesh(core_axis_name="c", subcore_axis_name="s")

    @functools.partial(
        pl.kernel,
        mesh=mesh,
        compiler_params=pltpu.CompilerParams(needs_layout_passes=False),
        out_type=jax.ShapeDtypeStruct((vocab // 2, 2 * dim), jnp.float32),
        scratch_types=[
            pltpu.VMEM((dim, _L), jnp.float32),
            pltpu.VMEM((dim, _L), jnp.float32),
            pltpu.VMEM((_L // 2, 2 * dim), jnp.float32),
            pltpu.VMEM((_L // 2, 2 * dim), jnp.float32),
            pltpu.VMEM((32, _L), jnp.float32),
            pltpu.SemaphoreType.DMA,
            pltpu.SemaphoreType.DMA,
            pltpu.SemaphoreType.DMA,
            pltpu.SemaphoreType.DMA,
        ],
    )
    def pack(wt_hbm, tail_hbm, w2_hbm, in0, in1, p0, p1, tl, sg0, sg1, sw0, sw1):
        wid = lax.axis_index("s") * _NC + lax.axis_index("c")
        ins = (in0, in1)
        outs = (p0, p1)
        sg = (sg0, sg1)
        sw = (sw0, sw1)
        rows_q = [_iota16() + 16 * q for q in range(4)]

        @pl.when(wid == 0)
        def _():
            pltpu.sync_copy(tail_hbm, tl)
            pltpu.sync_copy(tl, w2_hbm.at[pl.ds(n_cols * (_L // 2), 32)])

        def col_of(k):
            return k * _NW + wid

        def fire_loads(k, p):
            c = col_of(k)
            for g in range(dim // 8):
                pltpu.async_copy(
                    wt_hbm.at[pl.ds(8 * g, 8), pl.ds(c * _L, _L)],
                    ins[p].at[pl.ds(8 * g, 8)],
                    sg[p],
                )

        def wait_loads(k, p):
            c = col_of(k)
            for g in range(dim // 8):
                pltpu.make_async_copy(
                    wt_hbm.at[pl.ds(8 * g, 8), pl.ds(c * _L, _L)],
                    ins[p].at[pl.ds(8 * g, 8)],
                    sg[p],
                ).wait()

        def fire_store(k, p):
            c = col_of(k)
            pltpu.async_copy(
                outs[p], w2_hbm.at[pl.ds(c * (_L // 2), _L // 2)], sw[p]
            )

        def wait_store(k, p):
            c = col_of(k)
            pltpu.make_async_copy(
                outs[p], w2_hbm.at[pl.ds(c * (_L // 2), _L // 2)], sw[p]
            ).wait()

        def permute(p):
            @pl.loop(0, _L // 2, step=16)
            def _(rp0):
                for rr in range(16):
                    for u in range(2):
                        cols = jnp.full((16,), 2 * rr + u, jnp.int32) + 2 * rp0
                        for q in range(4):
                            v = plsc.load_gather(ins[p], [rows_q[q], cols])
                            outs[p][rp0 + rr, pl.ds(64 * u + 16 * q, 16)] = v

        @pl.when(col_of(0) < n_cols)
        def _():
            fire_loads(0, 0)

        @pl.loop(0, n_k2, step=2)
        def _(k0):
            for p in range(2):
                k = k0 + p
                ok = (k < n_k) & (col_of(k) < n_cols)
                nxt = (k + 1 < n_k) & (col_of(k + 1) < n_cols)
                prv = (k >= 1) & (k - 1 < n_k) & (col_of(k - 1) < n_cols)

                @pl.when(ok)
                def _():
                    wait_loads(k, p)

                @pl.when(nxt)
                def _():
                    fire_loads(k + 1, 1 - p)

                @pl.when(prv)
                def _():
                    wait_store(k - 1, 1 - p)

                @pl.when(ok)
                def _():
                    permute(p)
                    fire_store(k, p)

        if n_k % 2 == 0:
            last = n_k - 1

            @pl.when(col_of(last) < n_cols)
            def _():
                wait_store(last, last % 2)

    return pack


def _gather_call(batch, hist, dim, vocab):
    n = batch * hist
    n_blocks = n // _L
    bpw = n_blocks // _NW
    cpb = batch // _L
    mesh = plsc.VectorSubcoreMesh(core_axis_name="c", subcore_axis_name="s")

    @functools.partial(
        pl.kernel,
        mesh=mesh,
        compiler_params=pltpu.CompilerParams(needs_layout_passes=False),
        out_type=jax.ShapeDtypeStruct((hist, dim, batch), jnp.float32),
        scratch_types=[
            pltpu.VMEM((bpw, _L), jnp.int32),
            pltpu.VMEM((bpw, _L), jnp.int32),
            pltpu.VMEM((_L, 2 * dim), jnp.float32),
            pltpu.VMEM((_L, 2 * dim), jnp.float32),
            pltpu.VMEM((dim, _L), jnp.float32),
            pltpu.VMEM((dim, _L), jnp.float32),
            pltpu.SemaphoreType.DMA,
            pltpu.SemaphoreType.DMA,
            pltpu.SemaphoreType.DMA,
            pltpu.SemaphoreType.DMA,
        ],
    )
    def gat(idx_hbm, par_hbm, w2_hbm, out_hbm,
            idx_v, par_v, b0, b1, t0, t1, sg0, sg1, sw0, sw1):
        wid = lax.axis_index("s") * _NC + lax.axis_index("c")
        base = wid * bpw
        pltpu.sync_copy(idx_hbm.at[pl.ds(base, bpw)], idx_v)
        pltpu.sync_copy(par_hbm.at[pl.ds(base, bpw)], par_v)
        bufs = (b0, b1)
        tout = (t0, t1)
        sg = (sg0, sg1)
        sw = (sw0, sw1)
        rows_q = [_iota16() + 16 * q for q in range(8)]

        def fire_gather(jl, p):
            pltpu.async_copy(w2_hbm.at[idx_v.at[jl]], bufs[p], sg[p])

        def wait_gather(jl, p):
            pltpu.make_async_copy(w2_hbm.at[idx_v.at[jl]], bufs[p], sg[p]).wait()

        def fire_wb(jl, p):
            j = base + jl
            h = j // cpb
            c = j % cpb
            for g in range(dim // 8):
                pltpu.async_copy(
                    tout[p].at[pl.ds(8 * g, 8)],
                    out_hbm.at[h, pl.ds(8 * g, 8), pl.ds(c * _L, _L)],
                    sw[p],
                )

        def wait_wb(jl, p):
            j = base + jl
            h = j // cpb
            c = j % cpb
            for g in range(dim // 8):
                pltpu.make_async_copy(
                    tout[p].at[pl.ds(8 * g, 8)],
                    out_hbm.at[h, pl.ds(8 * g, 8), pl.ds(c * _L, _L)],
                    sw[p],
                ).wait()

        def permute(jl, p):
            colbase = []
            for q in range(8):
                par_q = par_v[jl, pl.ds(16 * q, 16)]
                colbase.append(par_q * dim)

            @pl.loop(0, dim, step=16)
            def _(d0):
                for dd in range(16):
                    for q in range(8):
                        v = plsc.load_gather(
                            bufs[p], [rows_q[q], colbase[q] + (d0 + dd)]
                        )
                        tout[p][d0 + dd, pl.ds(16 * q, 16)] = v

        fire_gather(0, 0)

        @pl.loop(0, bpw, step=2)
        def _(j0):
            for p in range(2):
                jl = j0 + p
                wait_gather(jl, p)

                @pl.when(jl + 1 < bpw)
                def _():
                    fire_gather(jl + 1, 1 - p)

                @pl.when(jl >= 2)
                def _():
                    wait_wb(jl - 2, p)

                permute(jl, p)
                fire_wb(jl, p)

        wait_wb(bpw - 2, (bpw - 2) % 2)
        wait_wb(bpw - 1, (bpw - 1) % 2)

    return gat


def kernel(x, weight):
    batch, hist = x.shape
    vocab, dim = weight.shape
    assert vocab % 2 == 0 and dim % 8 == 0

    wt = weight.T
    n_cols = vocab // _L
    tail = weight[n_cols * _L:, :].reshape(-1, 2 * dim)
    w2 = _pack_table_call(dim, vocab)(wt, tail)

    xt = x.T.astype(jnp.int32).reshape((batch * hist) // _L, _L)
    idx = jnp.right_shift(xt, 1)
    par = jnp.bitwise_and(xt, 1)
    out_t = _gather_call(batch, hist, dim, vocab)(idx, par, w2)
    return out_t.transpose(2, 0, 1)


# speedup vs baseline: 1.3347x; 1.3347x over previous
import functools

import jax
import jax.numpy as jnp
from jax import lax
from jax.experimental import pallas as pl
from jax.experimental.pallas import tpu as pltpu
from jax.experimental.pallas import tpu_sc as plsc

_INFO = plsc.get_sparse_core_info()
_NC = _INFO.num_cores
_NS = _INFO.num_subcores
_NW = _NC * _NS
_L = 128
_IOTA = tuple(range(16))


def _iota16():
    return jax.lax.iota(jnp.int32, 16)


def _pack_table_call(dim, vocab):
    n_cols = (vocab // _L)
    n_k = (n_cols + _NW - 1) // _NW
    n_k2 = n_k + (n_k % 2)
    mesh = plsc.VectorSubcoreMesh(core_axis_name="c", subcore_axis_name="s")

    @functools.partial(
        pl.kernel,
        mesh=mesh,
        compiler_params=pltpu.CompilerParams(needs_layout_passes=False),
        out_type=jax.ShapeDtypeStruct((vocab // 2, 2 * dim), jnp.float32),
        scratch_types=[
            pltpu.VMEM((dim, _L), jnp.float32),
            pltpu.VMEM((dim, _L), jnp.float32),
            pltpu.VMEM((_L // 2, 2 * dim), jnp.float32),
            pltpu.VMEM((_L // 2, 2 * dim), jnp.float32),
            pltpu.VMEM((32, _L), jnp.float32),
            pltpu.SemaphoreType.DMA,
            pltpu.SemaphoreType.DMA,
            pltpu.SemaphoreType.DMA,
            pltpu.SemaphoreType.DMA,
        ],
    )
    def pack(wt_hbm, tail_hbm, w2_hbm, in0, in1, p0, p1, tl, sg0, sg1, sw0, sw1):
        wid = lax.axis_index("s") * _NC + lax.axis_index("c")
        ins = (in0, in1)
        outs = (p0, p1)
        sg = (sg0, sg1)
        sw = (sw0, sw1)
        rows_q = [_iota16() + 16 * q for q in range(4)]

        @pl.when(wid == 0)
        def _():
            pltpu.sync_copy(tail_hbm, tl)
            pltpu.sync_copy(tl, w2_hbm.at[pl.ds(n_cols * (_L // 2), 32)])

        def col_of(k):
            return k * _NW + wid

        def fire_loads(k, p):
            c = col_of(k)
            for g in range(dim // 8):
                pltpu.async_copy(
                    wt_hbm.at[pl.ds(8 * g, 8), pl.ds(c * _L, _L)],
                    ins[p].at[pl.ds(8 * g, 8)],
                    sg[p],
                )

        def wait_loads(k, p):
            c = col_of(k)
            for g in range(dim // 8):
                pltpu.make_async_copy(
                    wt_hbm.at[pl.ds(8 * g, 8), pl.ds(c * _L, _L)],
                    ins[p].at[pl.ds(8 * g, 8)],
                    sg[p],
                ).wait()

        def fire_store(k, p):
            c = col_of(k)
            pltpu.async_copy(
                outs[p], w2_hbm.at[pl.ds(c * (_L // 2), _L // 2)], sw[p]
            )

        def wait_store(k, p):
            c = col_of(k)
            pltpu.make_async_copy(
                outs[p], w2_hbm.at[pl.ds(c * (_L // 2), _L // 2)], sw[p]
            ).wait()

        def permute(p):
            @pl.loop(0, _L // 2, step=16)
            def _(rp0):
                for rr in range(16):
                    vs = []
                    for u in range(2):
                        cols = jnp.full((16,), 2 * rr + u, jnp.int32) + 2 * rp0
                        for q in range(4):
                            vs.append(plsc.load_gather(ins[p], [rows_q[q], cols]))
                    for u in range(2):
                        for q in range(4):
                            outs[p][rp0 + rr, pl.ds(64 * u + 16 * q, 16)] = vs[
                                4 * u + q
                            ]

        @pl.when(col_of(0) < n_cols)
        def _():
            fire_loads(0, 0)

        @pl.loop(0, n_k2, step=2)
        def _(k0):
            for p in range(2):
                k = k0 + p
                ok = (k < n_k) & (col_of(k) < n_cols)
                nxt = (k + 1 < n_k) & (col_of(k + 1) < n_cols)
                prv = (k >= 1) & (k - 1 < n_k) & (col_of(k - 1) < n_cols)

                @pl.when(ok)
                def _():
                    wait_loads(k, p)

                @pl.when(nxt)
                def _():
                    fire_loads(k + 1, 1 - p)

                @pl.when(prv)
                def _():
                    wait_store(k - 1, 1 - p)

                @pl.when(ok)
                def _():
                    permute(p)
                    fire_store(k, p)

        if n_k % 2 == 0:
            last = n_k - 1

            @pl.when(col_of(last) < n_cols)
            def _():
                wait_store(last, last % 2)

    return pack


def _gather_call(batch, hist, dim, vocab):
    n = batch * hist
    n_blocks = n // _L
    bpw = n_blocks // _NW
    cpb = batch // _L
    mesh = plsc.VectorSubcoreMesh(core_axis_name="c", subcore_axis_name="s")

    @functools.partial(
        pl.kernel,
        mesh=mesh,
        compiler_params=pltpu.CompilerParams(needs_layout_passes=False),
        out_type=jax.ShapeDtypeStruct((hist, dim, batch), jnp.float32),
        scratch_types=[
            pltpu.VMEM((bpw, _L), jnp.int32),
            pltpu.VMEM((bpw, _L), jnp.int32),
            pltpu.VMEM((_L, 2 * dim), jnp.float32),
            pltpu.VMEM((_L, 2 * dim), jnp.float32),
            pltpu.VMEM((dim, _L), jnp.float32),
            pltpu.VMEM((dim, _L), jnp.float32),
            pltpu.SemaphoreType.DMA,
            pltpu.SemaphoreType.DMA,
            pltpu.SemaphoreType.DMA,
            pltpu.SemaphoreType.DMA,
        ],
    )
    def gat(idx_hbm, par_hbm, w2_hbm, out_hbm,
            idx_v, par_v, b0, b1, t0, t1, sg0, sg1, sw0, sw1):
        wid = lax.axis_index("s") * _NC + lax.axis_index("c")
        base = wid * bpw
        pltpu.sync_copy(idx_hbm.at[pl.ds(base, bpw)], idx_v)
        pltpu.sync_copy(par_hbm.at[pl.ds(base, bpw)], par_v)
        bufs = (b0, b1)
        tout = (t0, t1)
        sg = (sg0, sg1)
        sw = (sw0, sw1)
        rows_q = [_iota16() + 16 * q for q in range(8)]

        def fire_gather(jl, p):
            pltpu.async_copy(w2_hbm.at[idx_v.at[jl]], bufs[p], sg[p])

        def wait_gather(jl, p):
            pltpu.make_async_copy(w2_hbm.at[idx_v.at[jl]], bufs[p], sg[p]).wait()

        def fire_wb(jl, p):
            j = base + jl
            h = j // cpb
            c = j % cpb
            for g in range(dim // 8):
                pltpu.async_copy(
                    tout[p].at[pl.ds(8 * g, 8)],
                    out_hbm.at[h, pl.ds(8 * g, 8), pl.ds(c * _L, _L)],
                    sw[p],
                )

        def wait_wb(jl, p):
            j = base + jl
            h = j // cpb
            c = j % cpb
            for g in range(dim // 8):
                pltpu.make_async_copy(
                    tout[p].at[pl.ds(8 * g, 8)],
                    out_hbm.at[h, pl.ds(8 * g, 8), pl.ds(c * _L, _L)],
                    sw[p],
                ).wait()

        def permute(jl, p):
            colbase = []
            for q in range(8):
                par_q = par_v[jl, pl.ds(16 * q, 16)]
                colbase.append(par_q * dim)

            @pl.loop(0, dim, step=16)
            def _(d0):
                for dd in range(16):
                    vs = []
                    for q in range(8):
                        vs.append(
                            plsc.load_gather(
                                bufs[p], [rows_q[q], colbase[q] + (d0 + dd)]
                            )
                        )
                    for q in range(8):
                        tout[p][d0 + dd, pl.ds(16 * q, 16)] = vs[q]

        fire_gather(0, 0)

        @pl.loop(0, bpw, step=2)
        def _(j0):
            for p in range(2):
                jl = j0 + p
                wait_gather(jl, p)

                @pl.when(jl + 1 < bpw)
                def _():
                    fire_gather(jl + 1, 1 - p)

                @pl.when(jl >= 2)
                def _():
                    wait_wb(jl - 2, p)

                permute(jl, p)
                fire_wb(jl, p)

        wait_wb(bpw - 2, (bpw - 2) % 2)
        wait_wb(bpw - 1, (bpw - 1) % 2)

    return gat


def kernel(x, weight):
    batch, hist = x.shape
    vocab, dim = weight.shape
    assert vocab % 2 == 0 and dim % 8 == 0

    wt = weight.T
    n_cols = vocab // _L
    tail = weight[n_cols * _L:, :].reshape(-1, 2 * dim)
    w2 = _pack_table_call(dim, vocab)(wt, tail)

    xt = x.T.astype(jnp.int32).reshape((batch * hist) // _L, _L)
    idx = jnp.right_shift(xt, 1)
    par = jnp.bitwise_and(xt, 1)
    out_t = _gather_call(batch, hist, dim, vocab)(idx, par, w2)
    return out_t.transpose(2, 0, 1)
